# baseline (speedup 1.0000x reference)
import functools

import jax
import jax.numpy as jnp
from jax import lax
from jax.experimental import pallas as pl
from jax.experimental.pallas import tpu as pltpu
from jax.experimental.pallas import tpu_sc as plsc

N = 10000
E = 320000
D = 128
DEPTH_SIZE = 64
EPS = 1e-5

NC = 2
NS = 16
NW = NC * NS
EPT = E // NW
K = 80
NCHUNK = EPT // K
NBUF = 3
GITER = -(-NCHUNK // NBUF)
RCH = 80
NRCH = N // RCH
RITER = -(-NRCH // NS)

_mesh = plsc.VectorSubcoreMesh(core_axis_name="c", subcore_axis_name="s")


def _sc_agg_body(dtype, tab_hbm, idx_hbm, agg_hbm, px_hbm,
                 idx_all, agg_ring, *rest):
    bufs = rest[0:NBUF]
    gsem = rest[NBUF:2 * NBUF]
    ssem = rest[2 * NBUF:3 * NBUF]
    asem = rest[3 * NBUF:4 * NBUF]
    acc_sh = rest[4 * NBUF]
    c = lax.axis_index("c")
    s = lax.axis_index("s")

    def rows_foreach(fn):
        for i in range(RITER):
            rc = s + NS * i

            @pl.when(rc < NRCH)
            def _():
                fn(rc * RCH)

    if dtype.itemsize == 4:
        zvec = jnp.zeros((16,), dtype)

        def zrow(r, carry):
            for cc in range(D // 16):
                bufs[0][r, pl.ds(cc * 16, 16)] = zvec
            return carry

        lax.fori_loop(0, K, zrow, 0)
    else:
        zblk = jnp.zeros((2, 16), dtype)

        def zrow(r, carry):
            for cc in range(D // 16):
                bufs[0][pl.ds(r * 2, 2), pl.ds(cc * 16, 16)] = zblk
            return carry

        lax.fori_loop(0, K // 2, zrow, 0)
    rows_foreach(lambda r0: pltpu.sync_copy(bufs[0],
                                            acc_sh.at[pl.ds(r0, RCH)]))

    wid = c * NS + s
    pltpu.sync_copy(idx_hbm.at[wid], idx_all)
    plsc.subcore_barrier()

    def gstart(j, b):
        pltpu.async_copy(tab_hbm.at[idx_all.at[j]], bufs[b], gsem[b])

    def gwait(j, b):
        pltpu.make_async_copy(tab_hbm.at[idx_all.at[j]], bufs[b], gsem[b]).wait()

    def astart(j, b):
        pltpu.async_copy(agg_hbm.at[wid, j], agg_ring.at[b], asem[b])

    def await_(j, b):
        pltpu.make_async_copy(agg_hbm.at[wid, j], agg_ring.at[b], asem[b]).wait()

    def sstart(j, b):
        pltpu.async_copy(bufs[b], acc_sh.at[agg_ring.at[b]], ssem[b], add=True)

    def swait(j, b):
        pltpu.make_async_copy(bufs[b], acc_sh.at[agg_ring.at[b]], ssem[b]).wait()

    astart(0, 0)
    gstart(0, 0)

    def group(g, carry):
        for t in range(NBUF):
            i = NBUF * g + t
            tn = (t + 1) % NBUF

            @pl.when(i + 1 < NCHUNK)
            def _():
                astart(i + 1, tn)
                gstart(i + 1, tn)

            @pl.when(i < NCHUNK)
            def _():
                gwait(i, t)
                await_(i, t)
        return carry

    lax.fori_loop(0, GITER, group, 0)
    plsc.subcore_barrier()

    def write_chunk(r0):
        pltpu.sync_copy(acc_sh.at[pl.ds(r0, RCH)], px_hbm.at[c, pl.ds(r0, RCH)])

    rows_foreach(write_chunk)


def _make_sc_agg(dtype):
    dtype = jnp.dtype(dtype)
    return pl.kernel(
        functools.partial(_sc_agg_body, dtype),
        mesh=_mesh,
        out_type=[jax.ShapeDtypeStruct((NC, N, D), dtype)],
        scratch_types=(
            [pltpu.VMEM((NCHUNK, K), jnp.int32),
             pltpu.VMEM((NBUF, K), jnp.int32)]
            + [pltpu.VMEM((K, D), dtype)] * NBUF
            + [pltpu.SemaphoreType.DMA] * (3 * NBUF)
            + [pltpu.VMEM_SHARED((N, D), dtype)]
        ),
    )


_sc_agg = _make_sc_agg(jnp.float32)


def _dense_body(px_ref, pc_ref, x_ref, wm_ref, wc_ref, g_ref, b_ref, o_ref):
    ax = px_ref[0] + px_ref[1]
    cnt = (pc_ref[0].astype(jnp.float32) + pc_ref[1].astype(jnp.float32))
    h = jnp.dot(ax, wm_ref[...], preferred_element_type=jnp.float32)
    h = h + jnp.dot(cnt, wc_ref[...], preferred_element_type=jnp.float32)
    h = h + x_ref[...]
    mean = jnp.mean(h, axis=0, keepdims=True)
    ctr = h - mean
    var = jnp.mean(ctr * ctr, axis=0, keepdims=True)
    o = g_ref[...] * ctr * lax.rsqrt(var + EPS) + b_ref[...]
    o_ref[...] = jnp.maximum(o, 0.0)


_dense = pl.pallas_call(
    _dense_body,
    out_shape=jax.ShapeDtypeStruct((N, D), jnp.float32),
)


def kernel(data, edge, edge_feature, emb,
           W_msg0, b_msg0, W_edge0, b_edge0, gamma0, beta0,
           W_msg1, b_msg1, W_edge1, b_edge1, gamma1, beta1):
    agg = edge[0].reshape(NW, NCHUNK, K)
    src = edge[1].reshape(NW, NCHUNK, K)
    REP = 32
    feat = (edge_feature + DEPTH_SIZE * (jnp.arange(E, dtype=jnp.int32) % REP)
            ).reshape(NW, NCHUNK, K)
    onehot = jnp.tile(jnp.eye(DEPTH_SIZE, D, dtype=jnp.float32), (REP, 1))

    (pc,) = _sc_agg(onehot, feat, agg)
    (px0,) = _sc_agg(data, src, agg)

    def edge_mat(W_edge, b_msg, b_edge):
        m = jnp.dot(emb, W_edge) + (b_msg + b_edge)[None, :]
        return jnp.zeros((D, D), jnp.float32).at[:DEPTH_SIZE].set(m)

    x1 = _dense(px0, pc, data, W_msg0, edge_mat(W_edge0, b_msg0, b_edge0),
                gamma0.reshape(1, D), beta0.reshape(1, D))

    (px1,) = _sc_agg(x1, src, agg)

    out = _dense(px1, pc, x1, W_msg1, edge_mat(W_edge1, b_msg1, b_edge1),
                 gamma1.reshape(1, D), beta1.reshape(1, D))
    return out


# speedup vs baseline: 1.1298x; 1.1298x over previous
import functools

import jax
import jax.numpy as jnp
from jax import lax
from jax.experimental import pallas as pl
from jax.experimental.pallas import tpu as pltpu
from jax.experimental.pallas import tpu_sc as plsc

N = 10000
E = 320000
D = 128
DEPTH_SIZE = 64
EPS = 1e-5

NC = 2
NS = 16
NW = NC * NS
EPT = E // NW
K = 80
NCHUNK = EPT // K
NBUF = 3
GITER = -(-NCHUNK // NBUF)
RCH = 80
NRCH = N // RCH
RITER = -(-NRCH // NS)

_mesh = plsc.VectorSubcoreMesh(core_axis_name="c", subcore_axis_name="s")


def _sc_agg_body(dtype, tab_hbm, idx_hbm, agg_hbm, px_hbm,
                 idx_all, agg_ring, *rest):
    bufs = rest[0:NBUF]
    gsem = rest[NBUF:2 * NBUF]
    ssem = rest[2 * NBUF:3 * NBUF]
    asem = rest[3 * NBUF:4 * NBUF]
    acc_sh = rest[4 * NBUF]
    c = lax.axis_index("c")
    s = lax.axis_index("s")

    def rows_foreach(fn):
        for i in range(RITER):
            rc = s + NS * i

            @pl.when(rc < NRCH)
            def _():
                fn(rc * RCH)

    if dtype.itemsize == 4:
        zvec = jnp.zeros((16,), dtype)

        def zrow(r, carry):
            for cc in range(D // 16):
                bufs[0][r, pl.ds(cc * 16, 16)] = zvec
            return carry

        lax.fori_loop(0, K, zrow, 0)
    else:
        zblk = jnp.zeros((2, 16), dtype)

        def zrow(r, carry):
            for cc in range(D // 16):
                bufs[0][pl.ds(r * 2, 2), pl.ds(cc * 16, 16)] = zblk
            return carry

        lax.fori_loop(0, K // 2, zrow, 0)
    rows_foreach(lambda r0: pltpu.sync_copy(bufs[0],
                                            acc_sh.at[pl.ds(r0, RCH)]))

    wid = c * NS + s
    pltpu.sync_copy(idx_hbm.at[wid], idx_all)
    plsc.subcore_barrier()

    def gstart(j, b):
        pltpu.async_copy(tab_hbm.at[idx_all.at[j]], bufs[b], gsem[b])

    def gwait(j, b):
        pltpu.make_async_copy(tab_hbm.at[idx_all.at[j]], bufs[b], gsem[b]).wait()

    def astart(j, b):
        pltpu.async_copy(agg_hbm.at[wid, j], agg_ring.at[b], asem[b])

    def await_(j, b):
        pltpu.make_async_copy(agg_hbm.at[wid, j], agg_ring.at[b], asem[b]).wait()

    def sstart(j, b):
        pltpu.async_copy(bufs[b], acc_sh.at[agg_ring.at[b]], ssem[b], add=True)

    def swait(j, b):
        pltpu.make_async_copy(bufs[b], acc_sh.at[agg_ring.at[b]], ssem[b]).wait()

    astart(0, 0)
    gstart(0, 0)

    def group(g, carry):
        for t in range(NBUF):
            i = NBUF * g + t
            tn = (t + 1) % NBUF

            @pl.when((i >= NBUF - 1) & (i < NCHUNK))
            def _():
                swait(i - (NBUF - 1), tn)

            @pl.when(i + 1 < NCHUNK)
            def _():
                astart(i + 1, tn)
                gstart(i + 1, tn)

            @pl.when(i < NCHUNK)
            def _():
                gwait(i, t)
                await_(i, t)
                sstart(i, t)
        return carry

    lax.fori_loop(0, GITER, group, 0)
    for j in range(NCHUNK - (NBUF - 1), NCHUNK):
        swait(j, j % NBUF)
    plsc.subcore_barrier()

    def write_chunk(r0):
        pltpu.sync_copy(acc_sh.at[pl.ds(r0, RCH)], px_hbm.at[c, pl.ds(r0, RCH)])

    rows_foreach(write_chunk)


def _make_sc_agg(dtype):
    dtype = jnp.dtype(dtype)
    return pl.kernel(
        functools.partial(_sc_agg_body, dtype),
        mesh=_mesh,
        out_type=[jax.ShapeDtypeStruct((NC, N, D), dtype)],
        scratch_types=(
            [pltpu.VMEM((NCHUNK, K), jnp.int32),
             pltpu.VMEM((NBUF, K), jnp.int32)]
            + [pltpu.VMEM((K, D), dtype)] * NBUF
            + [pltpu.SemaphoreType.DMA] * (3 * NBUF)
            + [pltpu.VMEM_SHARED((N, D), dtype)]
        ),
    )


_sc_agg = _make_sc_agg(jnp.float32)

FR = 6
CGI = -(-NCHUNK // FR)


def _sc_cnt_body(feat_hbm, agg_hbm, pc_hbm, fring, agg_ring, *rest):
    bufs = rest[0:NBUF]
    ssem = rest[NBUF:2 * NBUF]
    asem = rest[2 * NBUF:3 * NBUF]
    fsem = rest[3 * NBUF:3 * NBUF + FR]
    acc_sh = rest[3 * NBUF + FR]
    c = lax.axis_index("c")
    s = lax.axis_index("s")

    def rows_foreach(fn):
        for i in range(RITER):
            rc = s + NS * i

            @pl.when(rc < NRCH)
            def _():
                fn(rc * RCH)

    zvec = jnp.zeros((16,), jnp.float32)
    ones = jnp.ones((16,), jnp.float32)
    lanes = lax.iota(jnp.int32, 16)

    def zrow(r, carry):
        for cc in range(D // 16):
            for b in range(NBUF):
                bufs[b][r, pl.ds(cc * 16, 16)] = zvec
        return carry

    lax.fori_loop(0, K, zrow, 0)
    rows_foreach(lambda r0: pltpu.sync_copy(bufs[0],
                                            acc_sh.at[pl.ds(r0, RCH)]))

    wid = c * NS + s
    plsc.subcore_barrier()

    def ffetch(j, fs):
        pltpu.async_copy(feat_hbm.at[wid, j], fring.at[fs], fsem[fs])

    def fwait(j, fs):
        pltpu.make_async_copy(feat_hbm.at[wid, j], fring.at[fs], fsem[fs]).wait()

    def clear_rows(fs, b):
        def body_g(g, carry):
            fv = fring[fs, pl.ds(g * 16, 16)]
            for k in range(16):
                f = fv[k]
                off = pl.multiple_of((f >> 4) * 16, 16)
                bufs[b][g * 16 + k, pl.ds(off, 16)] = zvec
            return carry

        lax.fori_loop(0, K // 16, body_g, 0)

    def set_rows(fs, b):
        def body_g(g, carry):
            fv = fring[fs, pl.ds(g * 16, 16)]
            for k in range(16):
                f = fv[k]
                v = jnp.where(lanes == (f & 15), ones, zvec)
                off = pl.multiple_of((f >> 4) * 16, 16)
                bufs[b][g * 16 + k, pl.ds(off, 16)] = v
            return carry

        lax.fori_loop(0, K // 16, body_g, 0)

    def astart(j, b):
        pltpu.async_copy(agg_hbm.at[wid, j], agg_ring.at[b], asem[b])

    def await_(j, b):
        pltpu.make_async_copy(agg_hbm.at[wid, j], agg_ring.at[b], asem[b]).wait()

    def sstart(j, b):
        pltpu.async_copy(bufs[b], acc_sh.at[agg_ring.at[b]], ssem[b], add=True)

    def swait(j, b):
        pltpu.make_async_copy(bufs[b], acc_sh.at[agg_ring.at[b]], ssem[b]).wait()

    ffetch(0, 0)
    ffetch(1, 1)
    astart(0, 0)
    fwait(0, 0)
    set_rows(0, 0)

    def group(g, carry):
        for t in range(FR):
            i = FR * g + t
            bn = (t + 1) % NBUF

            @pl.when((i >= NBUF - 1) & (i < NCHUNK))
            def _():
                swait(i - (NBUF - 1), bn)

            @pl.when(i + 2 < NCHUNK)
            def _():
                ffetch(i + 2, (t + 2) % FR)

            @pl.when((i + 1 < NCHUNK) & (i >= NBUF - 1))
            def _():
                clear_rows((t + 1 - NBUF) % FR, bn)

            @pl.when(i + 1 < NCHUNK)
            def _():
                astart(i + 1, bn)
                fwait(i + 1, (t + 1) % FR)
                set_rows((t + 1) % FR, bn)

            @pl.when(i < NCHUNK)
            def _():
                await_(i, t % NBUF)
                sstart(i, t % NBUF)
        return carry

    lax.fori_loop(0, CGI, group, 0)
    for j in range(NCHUNK - (NBUF - 1), NCHUNK):
        swait(j, j % NBUF)
    plsc.subcore_barrier()

    def write_chunk(r0):
        pltpu.sync_copy(acc_sh.at[pl.ds(r0, RCH)], pc_hbm.at[c, pl.ds(r0, RCH)])

    rows_foreach(write_chunk)


_sc_cnt = pl.kernel(
    _sc_cnt_body,
    mesh=_mesh,
    out_type=[jax.ShapeDtypeStruct((NC, N, D), jnp.float32)],
    scratch_types=(
        [pltpu.VMEM((FR, K), jnp.int32),
         pltpu.VMEM((NBUF, K), jnp.int32)]
        + [pltpu.VMEM((K, D), jnp.float32)] * NBUF
        + [pltpu.SemaphoreType.DMA] * (2 * NBUF + FR)
        + [pltpu.VMEM_SHARED((N, D), jnp.float32)]
    ),
)


def _dense_body(px_ref, pc_ref, x_ref, wm_ref, wc_ref, g_ref, b_ref, o_ref):
    ax = px_ref[0] + px_ref[1]
    cnt = (pc_ref[0].astype(jnp.float32) + pc_ref[1].astype(jnp.float32))
    h = jnp.dot(ax, wm_ref[...], preferred_element_type=jnp.float32)
    h = h + jnp.dot(cnt, wc_ref[...], preferred_element_type=jnp.float32)
    h = h + x_ref[...]
    mean = jnp.mean(h, axis=0, keepdims=True)
    ctr = h - mean
    var = jnp.mean(ctr * ctr, axis=0, keepdims=True)
    o = g_ref[...] * ctr * lax.rsqrt(var + EPS) + b_ref[...]
    o_ref[...] = jnp.maximum(o, 0.0)


_dense = pl.pallas_call(
    _dense_body,
    out_shape=jax.ShapeDtypeStruct((N, D), jnp.float32),
)


def kernel(data, edge, edge_feature, emb,
           W_msg0, b_msg0, W_edge0, b_edge0, gamma0, beta0,
           W_msg1, b_msg1, W_edge1, b_edge1, gamma1, beta1):
    agg = edge[0].reshape(NW, NCHUNK, K)
    src = edge[1].reshape(NW, NCHUNK, K)
    feat = edge_feature.reshape(NW, NCHUNK, K)

    (pc,) = _sc_cnt(feat, agg)
    (px0,) = _sc_agg(data, src, agg)

    def edge_mat(W_edge, b_msg, b_edge):
        m = jnp.dot(emb, W_edge) + (b_msg + b_edge)[None, :]
        return jnp.zeros((D, D), jnp.float32).at[:DEPTH_SIZE].set(m)

    x1 = _dense(px0, pc, data, W_msg0, edge_mat(W_edge0, b_msg0, b_edge0),
                gamma0.reshape(1, D), beta0.reshape(1, D))

    (px1,) = _sc_agg(x1, src, agg)

    out = _dense(px1, pc, x1, W_msg1, edge_mat(W_edge1, b_msg1, b_edge1),
                 gamma1.reshape(1, D), beta1.reshape(1, D))
    return out


# speedup vs baseline: 1.1526x; 1.0201x over previous
import functools

import jax
import jax.numpy as jnp
from jax import lax
from jax.experimental import pallas as pl
from jax.experimental.pallas import tpu as pltpu
from jax.experimental.pallas import tpu_sc as plsc

N = 10000
E = 320000
D = 128
DEPTH_SIZE = 64
EPS = 1e-5

NC = 2
NS = 16
NW = NC * NS
EPT = E // NW
K = 80
NCHUNK = EPT // K
NBUF = 3
GITER = -(-NCHUNK // NBUF)
RCH = 80
NRCH = N // RCH
RITER = -(-NRCH // NS)

_mesh = plsc.VectorSubcoreMesh(core_axis_name="c", subcore_axis_name="s")


def _sc_agg_body(dtype, tab_hbm, idx_hbm, agg_hbm, px_hbm,
                 idx_all, agg_ring, *rest):
    bufs = rest[0:NBUF]
    gsem = rest[NBUF:2 * NBUF]
    ssem = rest[2 * NBUF:3 * NBUF]
    asem = rest[3 * NBUF:4 * NBUF]
    acc_sh = rest[4 * NBUF]
    c = lax.axis_index("c")
    s = lax.axis_index("s")

    def rows_foreach(fn):
        for i in range(RITER):
            rc = s + NS * i

            @pl.when(rc < NRCH)
            def _():
                fn(rc * RCH)

    if dtype.itemsize == 4:
        zvec = jnp.zeros((16,), dtype)

        def zrow(r, carry):
            for cc in range(D // 16):
                bufs[0][r, pl.ds(cc * 16, 16)] = zvec
            return carry

        lax.fori_loop(0, K, zrow, 0)
    else:
        zblk = jnp.zeros((2, 16), dtype)

        def zrow(r, carry):
            for cc in range(D // 16):
                bufs[0][pl.ds(r * 2, 2), pl.ds(cc * 16, 16)] = zblk
            return carry

        lax.fori_loop(0, K // 2, zrow, 0)
    rows_foreach(lambda r0: pltpu.sync_copy(bufs[0],
                                            acc_sh.at[pl.ds(r0, RCH)]))

    wid = c * NS + s
    pltpu.sync_copy(idx_hbm.at[wid], idx_all)
    plsc.subcore_barrier()

    def gstart(j, b):
        pltpu.async_copy(tab_hbm.at[idx_all.at[j]], bufs[b], gsem[b])

    def gwait(j, b):
        pltpu.make_async_copy(tab_hbm.at[idx_all.at[j]], bufs[b], gsem[b]).wait()

    def astart(j, b):
        pltpu.async_copy(agg_hbm.at[wid, j], agg_ring.at[b], asem[b])

    def await_(j, b):
        pltpu.make_async_copy(agg_hbm.at[wid, j], agg_ring.at[b], asem[b]).wait()

    def sstart(j, b):
        pltpu.async_copy(bufs[b], acc_sh.at[agg_ring.at[b]], ssem[b], add=True)

    def swait(j, b):
        pltpu.make_async_copy(bufs[b], acc_sh.at[agg_ring.at[b]], ssem[b]).wait()

    astart(0, 0)
    gstart(0, 0)
    astart(1, 1)
    gstart(1, 1)

    def group(g, carry):
        for t in range(NBUF):
            i = NBUF * g + t
            tn = (t + 2) % NBUF

            @pl.when((i >= 1) & (i < NCHUNK))
            def _():
                swait(i - 1, tn)

            @pl.when(i + 2 < NCHUNK)
            def _():
                astart(i + 2, tn)
                gstart(i + 2, tn)

            @pl.when(i < NCHUNK)
            def _():
                gwait(i, t)
                await_(i, t)
                sstart(i, t)
        return carry

    lax.fori_loop(0, GITER, group, 0)
    swait(NCHUNK - 1, (NCHUNK - 1) % NBUF)
    plsc.subcore_barrier()

    def write_chunk(r0):
        pltpu.sync_copy(acc_sh.at[pl.ds(r0, RCH)], px_hbm.at[c, pl.ds(r0, RCH)])

    rows_foreach(write_chunk)


def _make_sc_agg(dtype):
    dtype = jnp.dtype(dtype)
    return pl.kernel(
        functools.partial(_sc_agg_body, dtype),
        mesh=_mesh,
        out_type=[jax.ShapeDtypeStruct((NC, N, D), dtype)],
        scratch_types=(
            [pltpu.VMEM((NCHUNK, K), jnp.int32),
             pltpu.VMEM((NBUF, K), jnp.int32)]
            + [pltpu.VMEM((K, D), dtype)] * NBUF
            + [pltpu.SemaphoreType.DMA] * (3 * NBUF)
            + [pltpu.VMEM_SHARED((N, D), dtype)]
        ),
    )


_sc_agg = _make_sc_agg(jnp.float32)

FR = 6
CGI = -(-NCHUNK // FR)


def _sc_cnt_body(feat_hbm, agg_hbm, pc_hbm, fring, agg_ring, *rest):
    bufs = rest[0:NBUF]
    ssem = rest[NBUF:2 * NBUF]
    asem = rest[2 * NBUF:3 * NBUF]
    fsem = rest[3 * NBUF:3 * NBUF + FR]
    acc_sh = rest[3 * NBUF + FR]
    c = lax.axis_index("c")
    s = lax.axis_index("s")

    def rows_foreach(fn):
        for i in range(RITER):
            rc = s + NS * i

            @pl.when(rc < NRCH)
            def _():
                fn(rc * RCH)

    zvec = jnp.zeros((16,), jnp.float32)
    ones = jnp.ones((16,), jnp.float32)
    lanes = lax.iota(jnp.int32, 16)

    def zrow(r, carry):
        for cc in range(D // 16):
            for b in range(NBUF):
                bufs[b][r, pl.ds(cc * 16, 16)] = zvec
        return carry

    lax.fori_loop(0, K, zrow, 0)
    rows_foreach(lambda r0: pltpu.sync_copy(bufs[0],
                                            acc_sh.at[pl.ds(r0, RCH)]))

    wid = c * NS + s
    plsc.subcore_barrier()

    def ffetch(j, fs):
        pltpu.async_copy(feat_hbm.at[wid, j], fring.at[fs], fsem[fs])

    def fwait(j, fs):
        pltpu.make_async_copy(feat_hbm.at[wid, j], fring.at[fs], fsem[fs]).wait()

    def clear_rows(fs, b):
        def body_g(g, carry):
            fv = fring[fs, pl.ds(g * 16, 16)]
            for k in range(16):
                f = fv[k]
                off = pl.multiple_of((f >> 4) * 16, 16)
                bufs[b][g * 16 + k, pl.ds(off, 16)] = zvec
            return carry

        lax.fori_loop(0, K // 16, body_g, 0)

    def set_rows(fs, b):
        def body_g(g, carry):
            fv = fring[fs, pl.ds(g * 16, 16)]
            for k in range(16):
                f = fv[k]
                v = jnp.where(lanes == (f & 15), ones, zvec)
                off = pl.multiple_of((f >> 4) * 16, 16)
                bufs[b][g * 16 + k, pl.ds(off, 16)] = v
            return carry

        lax.fori_loop(0, K // 16, body_g, 0)

    def astart(j, b):
        pltpu.async_copy(agg_hbm.at[wid, j], agg_ring.at[b], asem[b])

    def await_(j, b):
        pltpu.make_async_copy(agg_hbm.at[wid, j], agg_ring.at[b], asem[b]).wait()

    def sstart(j, b):
        pltpu.async_copy(bufs[b], acc_sh.at[agg_ring.at[b]], ssem[b], add=True)

    def swait(j, b):
        pltpu.make_async_copy(bufs[b], acc_sh.at[agg_ring.at[b]], ssem[b]).wait()

    ffetch(0, 0)
    ffetch(1, 1)
    astart(0, 0)
    fwait(0, 0)
    set_rows(0, 0)

    def group(g, carry):
        for t in range(FR):
            i = FR * g + t
            bn = (t + 1) % NBUF

            @pl.when((i >= NBUF - 1) & (i < NCHUNK))
            def _():
                swait(i - (NBUF - 1), bn)

            @pl.when(i + 2 < NCHUNK)
            def _():
                ffetch(i + 2, (t + 2) % FR)

            @pl.when((i + 1 < NCHUNK) & (i >= NBUF - 1))
            def _():
                clear_rows((t + 1 - NBUF) % FR, bn)

            @pl.when(i + 1 < NCHUNK)
            def _():
                astart(i + 1, bn)
                fwait(i + 1, (t + 1) % FR)
                set_rows((t + 1) % FR, bn)

            @pl.when(i < NCHUNK)
            def _():
                await_(i, t % NBUF)
                sstart(i, t % NBUF)
        return carry

    lax.fori_loop(0, CGI, group, 0)
    for j in range(NCHUNK - (NBUF - 1), NCHUNK):
        swait(j, j % NBUF)
    plsc.subcore_barrier()

    def write_chunk(r0):
        pltpu.sync_copy(acc_sh.at[pl.ds(r0, RCH)], pc_hbm.at[c, pl.ds(r0, RCH)])

    rows_foreach(write_chunk)


_sc_cnt = pl.kernel(
    _sc_cnt_body,
    mesh=_mesh,
    out_type=[jax.ShapeDtypeStruct((NC, N, D), jnp.float32)],
    scratch_types=(
        [pltpu.VMEM((FR, K), jnp.int32),
         pltpu.VMEM((NBUF, K), jnp.int32)]
        + [pltpu.VMEM((K, D), jnp.float32)] * NBUF
        + [pltpu.SemaphoreType.DMA] * (2 * NBUF + FR)
        + [pltpu.VMEM_SHARED((N, D), jnp.float32)]
    ),
)


def _dense_body(px_ref, pc_ref, x_ref, wm_ref, wc_ref, g_ref, b_ref, o_ref):
    ax = px_ref[0] + px_ref[1]
    cnt = (pc_ref[0].astype(jnp.float32) + pc_ref[1].astype(jnp.float32))
    h = jnp.dot(ax, wm_ref[...], preferred_element_type=jnp.float32)
    h = h + jnp.dot(cnt, wc_ref[...], preferred_element_type=jnp.float32)
    h = h + x_ref[...]
    mean = jnp.mean(h, axis=0, keepdims=True)
    ctr = h - mean
    var = jnp.mean(ctr * ctr, axis=0, keepdims=True)
    o = g_ref[...] * ctr * lax.rsqrt(var + EPS) + b_ref[...]
    o_ref[...] = jnp.maximum(o, 0.0)


_dense = pl.pallas_call(
    _dense_body,
    out_shape=jax.ShapeDtypeStruct((N, D), jnp.float32),
)


def kernel(data, edge, edge_feature, emb,
           W_msg0, b_msg0, W_edge0, b_edge0, gamma0, beta0,
           W_msg1, b_msg1, W_edge1, b_edge1, gamma1, beta1):
    agg = edge[0].reshape(NW, NCHUNK, K)
    src = edge[1].reshape(NW, NCHUNK, K)
    feat = edge_feature.reshape(NW, NCHUNK, K)

    (pc,) = _sc_cnt(feat, agg)
    (px0,) = _sc_agg(data, src, agg)

    def edge_mat(W_edge, b_msg, b_edge):
        m = jnp.dot(emb, W_edge) + (b_msg + b_edge)[None, :]
        return jnp.zeros((D, D), jnp.float32).at[:DEPTH_SIZE].set(m)

    x1 = _dense(px0, pc, data, W_msg0, edge_mat(W_edge0, b_msg0, b_edge0),
                gamma0.reshape(1, D), beta0.reshape(1, D))

    (px1,) = _sc_agg(x1, src, agg)

    out = _dense(px1, pc, x1, W_msg1, edge_mat(W_edge1, b_msg1, b_edge1),
                 gamma1.reshape(1, D), beta1.reshape(1, D))
    return out


# speedup vs baseline: 1.1713x; 1.0163x over previous
import functools

import jax
import jax.numpy as jnp
from jax import lax
from jax.experimental import pallas as pl
from jax.experimental.pallas import tpu as pltpu
from jax.experimental.pallas import tpu_sc as plsc

N = 10000
E = 320000
D = 128
DEPTH_SIZE = 64
EPS = 1e-5

NC = 2
NS = 16
NW = NC * NS
EPT = E // NW
K = 80
NCHUNK = EPT // K
NBUF = 3
GITER = -(-NCHUNK // NBUF)
RCH = 80
NRCH = N // RCH
RITER = -(-NRCH // NS)

_mesh = plsc.VectorSubcoreMesh(core_axis_name="c", subcore_axis_name="s")


def _sc_agg_body(dtype, tab_hbm, idx_hbm, agg_hbm, px_hbm,
                 idx_all, agg_ring, *rest):
    bufs = rest[0:NBUF]
    gsem = rest[NBUF:2 * NBUF]
    ssem = rest[2 * NBUF:3 * NBUF]
    asem = rest[3 * NBUF:4 * NBUF]
    acc_sh = rest[4 * NBUF]
    c = lax.axis_index("c")
    s = lax.axis_index("s")

    def rows_foreach(fn):
        for i in range(RITER):
            rc = s + NS * i

            @pl.when(rc < NRCH)
            def _():
                fn(rc * RCH)

    if dtype.itemsize == 4:
        zvec = jnp.zeros((16,), dtype)

        def zrow(r, carry):
            for cc in range(D // 16):
                bufs[0][r, pl.ds(cc * 16, 16)] = zvec
            return carry

        lax.fori_loop(0, K, zrow, 0)
    else:
        zblk = jnp.zeros((2, 16), dtype)

        def zrow(r, carry):
            for cc in range(D // 16):
                bufs[0][pl.ds(r * 2, 2), pl.ds(cc * 16, 16)] = zblk
            return carry

        lax.fori_loop(0, K // 2, zrow, 0)
    rows_foreach(lambda r0: pltpu.sync_copy(bufs[0],
                                            acc_sh.at[pl.ds(r0, RCH)]))

    wid = c * NS + s
    pltpu.sync_copy(idx_hbm.at[wid], idx_all)
    plsc.subcore_barrier()

    def gstart(j, b):
        pltpu.async_copy(tab_hbm.at[idx_all.at[j]], bufs[b], gsem[b])

    def gwait(j, b):
        pltpu.make_async_copy(tab_hbm.at[idx_all.at[j]], bufs[b], gsem[b]).wait()

    def astart(j, b):
        pltpu.async_copy(agg_hbm.at[wid, j], agg_ring.at[b], asem[b])

    def await_(j, b):
        pltpu.make_async_copy(agg_hbm.at[wid, j], agg_ring.at[b], asem[b]).wait()

    def sstart(j, b):
        pltpu.async_copy(bufs[b], acc_sh.at[agg_ring.at[b]], ssem[b], add=True)

    def swait(j, b):
        pltpu.make_async_copy(bufs[b], acc_sh.at[agg_ring.at[b]], ssem[b]).wait()

    astart(0, 0)
    gstart(0, 0)
    astart(1, 1)
    gstart(1, 1)

    def group(g, carry):
        for t in range(NBUF):
            i = NBUF * g + t
            tn = (t + 2) % NBUF

            @pl.when((i >= 1) & (i < NCHUNK))
            def _():
                swait(i - 1, tn)

            @pl.when(i + 2 < NCHUNK)
            def _():
                astart(i + 2, tn)
                gstart(i + 2, tn)

            @pl.when(i < NCHUNK)
            def _():
                gwait(i, t)
                await_(i, t)
                sstart(i, t)
        return carry

    lax.fori_loop(0, GITER, group, 0)
    swait(NCHUNK - 1, (NCHUNK - 1) % NBUF)
    plsc.subcore_barrier()

    def write_chunk(r0):
        pltpu.sync_copy(acc_sh.at[pl.ds(r0, RCH)], px_hbm.at[c, pl.ds(r0, RCH)])

    rows_foreach(write_chunk)


def _make_sc_agg(dtype):
    dtype = jnp.dtype(dtype)
    return pl.kernel(
        functools.partial(_sc_agg_body, dtype),
        mesh=_mesh,
        out_type=[jax.ShapeDtypeStruct((NC, N, D), dtype)],
        scratch_types=(
            [pltpu.VMEM((NCHUNK, K), jnp.int32),
             pltpu.VMEM((NBUF, K), jnp.int32)]
            + [pltpu.VMEM((K, D), dtype)] * NBUF
            + [pltpu.SemaphoreType.DMA] * (3 * NBUF)
            + [pltpu.VMEM_SHARED((N, D), dtype)]
        ),
    )


_sc_agg = _make_sc_agg(jnp.float32)

FR = 6
CGI = -(-NCHUNK // FR)


def _sc_cnt_body(tab_hbm, src_hbm, feat_hbm, agg_hbm, px_hbm, pc_hbm,
                 idx_all, fring, agg_ring, *rest):
    bufs = rest[0:NBUF]
    gsem = rest[NBUF:2 * NBUF]
    ssem = rest[2 * NBUF:3 * NBUF]
    asem = rest[3 * NBUF:4 * NBUF]
    fsem = rest[4 * NBUF:4 * NBUF + FR]
    stgsem = rest[4 * NBUF + FR]
    acc_sh = rest[4 * NBUF + FR + 1]
    c = lax.axis_index("c")
    s = lax.axis_index("s")

    def rows_foreach(fn):
        for i in range(RITER):
            rc = s + NS * i

            @pl.when(rc < NRCH)
            def _():
                fn(rc * RCH)

    zvec = jnp.zeros((16,), jnp.float32)
    ones = jnp.ones((16,), jnp.float32)
    lanes = lax.iota(jnp.int32, 16)

    def zrow(r, carry):
        for cc in range(D // 16):
            for b in range(NBUF):
                bufs[b][r, pl.ds(cc * 16, 16)] = zvec
        return carry

    def zrow0(r, carry):
        for cc in range(D // 16):
            bufs[0][r, pl.ds(cc * 16, 16)] = zvec
        return carry

    lax.fori_loop(0, K, zrow, 0)
    rows_foreach(lambda r0: pltpu.sync_copy(bufs[0],
                                            acc_sh.at[pl.ds(r0, RCH)]))

    wid = c * NS + s
    pltpu.async_copy(src_hbm.at[wid], idx_all, stgsem)
    plsc.subcore_barrier()

    def ffetch(j, fs):
        pltpu.async_copy(feat_hbm.at[wid, j], fring.at[fs], fsem[fs])

    def fwait(j, fs):
        pltpu.make_async_copy(feat_hbm.at[wid, j], fring.at[fs], fsem[fs]).wait()

    def clear_set_rows(fso, fsn, b, with_clear):
        def body_g(g, carry):
            if with_clear:
                fvo = fring[fso, pl.ds(g * 16, 16)]
            fvn = fring[fsn, pl.ds(g * 16, 16)]
            for k in range(16):
                if with_clear:
                    fo = fvo[k]
                    offo = pl.multiple_of((fo >> 4) * 16, 16)
                    bufs[b][g * 16 + k, pl.ds(offo, 16)] = zvec
                fn = fvn[k]
                v = jnp.where(lanes == (fn & 15), ones, zvec)
                offn = pl.multiple_of((fn >> 4) * 16, 16)
                bufs[b][g * 16 + k, pl.ds(offn, 16)] = v
            return carry

        lax.fori_loop(0, K // 16, body_g, 0)

    def astart(j, b):
        pltpu.async_copy(agg_hbm.at[wid, j], agg_ring.at[b], asem[b])

    def await_(j, b):
        pltpu.make_async_copy(agg_hbm.at[wid, j], agg_ring.at[b], asem[b]).wait()

    def sstart(j, b):
        pltpu.async_copy(bufs[b], acc_sh.at[agg_ring.at[b]], ssem[b], add=True)

    def swait(j, b):
        pltpu.make_async_copy(bufs[b], acc_sh.at[agg_ring.at[b]], ssem[b]).wait()

    ffetch(0, 0)
    ffetch(1, 1)
    astart(0, 0)
    fwait(0, 0)
    clear_set_rows(0, 0, 0, False)

    def group(g, carry):
        for t in range(FR):
            i = FR * g + t
            bn = (t + 1) % NBUF

            @pl.when((i >= NBUF - 1) & (i < NCHUNK))
            def _():
                swait(i - (NBUF - 1), bn)

            @pl.when(i + 2 < NCHUNK)
            def _():
                ffetch(i + 2, (t + 2) % FR)

            @pl.when((i + 1 < NCHUNK) & (i >= NBUF - 1))
            def _():
                astart(i + 1, bn)
                fwait(i + 1, (t + 1) % FR)
                clear_set_rows((t + 1 - NBUF) % FR, (t + 1) % FR, bn, True)

            @pl.when((i + 1 < NCHUNK) & (i < NBUF - 1))
            def _():
                astart(i + 1, bn)
                fwait(i + 1, (t + 1) % FR)
                clear_set_rows((t + 1) % FR, (t + 1) % FR, bn, False)

            @pl.when(i < NCHUNK)
            def _():
                await_(i, t % NBUF)
                sstart(i, t % NBUF)
        return carry

    lax.fori_loop(0, CGI, group, 0)
    for j in range(NCHUNK - (NBUF - 1), NCHUNK):
        swait(j, j % NBUF)
    plsc.subcore_barrier()

    def write_pc(r0):
        pltpu.sync_copy(acc_sh.at[pl.ds(r0, RCH)], pc_hbm.at[c, pl.ds(r0, RCH)])

    rows_foreach(write_pc)

    lax.fori_loop(0, K, zrow0, 0)
    rows_foreach(lambda r0: pltpu.sync_copy(bufs[0],
                                            acc_sh.at[pl.ds(r0, RCH)]))
    pltpu.make_async_copy(src_hbm.at[wid], idx_all, stgsem).wait()
    plsc.subcore_barrier()

    def gstart(j, b):
        pltpu.async_copy(tab_hbm.at[idx_all.at[j]], bufs[b], gsem[b])

    def gwait(j, b):
        pltpu.make_async_copy(tab_hbm.at[idx_all.at[j]], bufs[b], gsem[b]).wait()

    astart(0, 0)
    gstart(0, 0)
    astart(1, 1)
    gstart(1, 1)

    def group_b(g, carry):
        for t in range(NBUF):
            i = NBUF * g + t
            tn = (t + 2) % NBUF

            @pl.when((i >= 1) & (i < NCHUNK))
            def _():
                swait(i - 1, tn)

            @pl.when(i + 2 < NCHUNK)
            def _():
                astart(i + 2, tn)
                gstart(i + 2, tn)

            @pl.when(i < NCHUNK)
            def _():
                gwait(i, t)
                await_(i, t)
                sstart(i, t)
        return carry

    lax.fori_loop(0, GITER, group_b, 0)
    swait(NCHUNK - 1, (NCHUNK - 1) % NBUF)
    plsc.subcore_barrier()

    def write_px(r0):
        pltpu.sync_copy(acc_sh.at[pl.ds(r0, RCH)], px_hbm.at[c, pl.ds(r0, RCH)])

    rows_foreach(write_px)


_sc_cnt = pl.kernel(
    _sc_cnt_body,
    mesh=_mesh,
    out_type=[jax.ShapeDtypeStruct((NC, N, D), jnp.float32),
              jax.ShapeDtypeStruct((NC, N, D), jnp.float32)],
    scratch_types=(
        [pltpu.VMEM((NCHUNK, K), jnp.int32),
         pltpu.VMEM((FR, K), jnp.int32),
         pltpu.VMEM((NBUF, K), jnp.int32)]
        + [pltpu.VMEM((K, D), jnp.float32)] * NBUF
        + [pltpu.SemaphoreType.DMA] * (3 * NBUF + FR + 1)
        + [pltpu.VMEM_SHARED((N, D), jnp.float32)]
    ),
)


def _dense_body(px_ref, pc_ref, x_ref, wm_ref, wc_ref, g_ref, b_ref, o_ref):
    ax = px_ref[0] + px_ref[1]
    cnt = (pc_ref[0].astype(jnp.float32) + pc_ref[1].astype(jnp.float32))
    h = jnp.dot(ax, wm_ref[...], preferred_element_type=jnp.float32)
    h = h + jnp.dot(cnt, wc_ref[...], preferred_element_type=jnp.float32)
    h = h + x_ref[...]
    mean = jnp.mean(h, axis=0, keepdims=True)
    ctr = h - mean
    var = jnp.mean(ctr * ctr, axis=0, keepdims=True)
    o = g_ref[...] * ctr * lax.rsqrt(var + EPS) + b_ref[...]
    o_ref[...] = jnp.maximum(o, 0.0)


_dense = pl.pallas_call(
    _dense_body,
    out_shape=jax.ShapeDtypeStruct((N, D), jnp.float32),
)


def kernel(data, edge, edge_feature, emb,
           W_msg0, b_msg0, W_edge0, b_edge0, gamma0, beta0,
           W_msg1, b_msg1, W_edge1, b_edge1, gamma1, beta1):
    agg = edge[0].reshape(NW, NCHUNK, K)
    src = edge[1].reshape(NW, NCHUNK, K)
    feat = edge_feature.reshape(NW, NCHUNK, K)

    px0, pc = _sc_cnt(data, src, feat, agg)

    def edge_mat(W_edge, b_msg, b_edge):
        m = jnp.dot(emb, W_edge) + (b_msg + b_edge)[None, :]
        return jnp.zeros((D, D), jnp.float32).at[:DEPTH_SIZE].set(m)

    x1 = _dense(px0, pc, data, W_msg0, edge_mat(W_edge0, b_msg0, b_edge0),
                gamma0.reshape(1, D), beta0.reshape(1, D))

    (px1,) = _sc_agg(x1, src, agg)

    out = _dense(px1, pc, x1, W_msg1, edge_mat(W_edge1, b_msg1, b_edge1),
                 gamma1.reshape(1, D), beta1.reshape(1, D))
    return out
